# baseline (speedup 1.0000x reference)
import jax
import jax.numpy as jnp
from jax import lax
from jax.experimental import pallas as pl
from jax.experimental.pallas import tpu as pltpu
from jax.experimental.pallas import tpu_sc as plsc
from jax._src.pallas import mpmd as _mpmd

NUM_NODES = 100000
HIDDEN = 64
INPUT_DIM = 128
BATCH = 16384
PADW = 128

NW = 32
LANES = 16
B_PER_W = BATCH // NW
NVREG = BATCH // LANES
LOCAL_PAD = 3136
NLOCV = LOCAL_PAD // LANES
POS_BITS = 14
POS_MASK = (1 << POS_BITS) - 1
SENTINEL = 2**31 - 1
CHUNK = 128
NBUF = 4
GROUP = NBUF * CHUNK
NCH_MAX = 32
CAP = NCH_MAX * CHUNK


def _shift_up(v):
  ii = lax.iota(jnp.int32, LANES)
  ind = jnp.minimum(ii + 1, LANES - 1)
  return lax.gather(
      v, ind[:, None],
      dimension_numbers=lax.GatherDimensionNumbers(
          offset_dims=(), collapsed_slice_dims=(0,), start_index_map=(0,)),
      slice_sizes=(1,),
      mode=lax.GatherScatterMode.PROMISE_IN_BOUNDS)


_SC_PARAMS = pltpu.CompilerParams(needs_layout_passes=False,
                                  use_tc_tiling_on_sc=True)


def _mesh():
  return plsc.VectorSubcoreMesh(core_axis_name="c", subcore_axis_name="s",
                                num_cores=2, num_subcores=16)


def _k1_body(idx_hbm, tbl_hbm, m1g_hbm, lv_hbm, tv_hbm, nn_hbm,
             idxv, rows, idx_all, s_tbl, l_v, t_v, nv, gsem):
  wid = lax.axis_index("s") * 2 + lax.axis_index("c")
  ii = lax.iota(jnp.int32, LANES)

  pltpu.sync_copy(idx_hbm.at[pl.ds(wid * B_PER_W, B_PER_W)], idxv)
  descs = []
  for k in range(B_PER_W // CHUNK):
    descs.append(pltpu.async_copy(
        tbl_hbm.at[idxv.at[pl.ds(k * CHUNK, CHUNK)]],
        rows.at[pl.ds(k * CHUNK, CHUNK)], gsem))

  pltpu.sync_copy(idx_hbm, idx_all)

  neg1 = jnp.full((LANES,), -1, jnp.int32)
  def init_body(k, _):
    s_tbl[pl.ds(k * LANES, LANES)] = neg1
    return 0
  lax.fori_loop(0, NLOCV, init_body, 0, unroll=8)

  def scan_body(j, _):
    v = idx_all[pl.ds(j * LANES, LANES)]
    mask = (v & (NW - 1)) == wid
    local = lax.shift_right_logical(v, 5)
    pos = j * LANES + ii
    comb = jnp.where(mask, (local << POS_BITS) | pos, SENTINEL)
    csort, _ = plsc.sort_key_val(comb, comb)
    nxt = _shift_up(csort)
    loc_s = lax.shift_right_logical(csort, POS_BITS)
    nxt_s = lax.shift_right_logical(nxt, POS_BITS)
    win = ((loc_s != nxt_s) | (ii == LANES - 1)) & (csort != SENTINEL)
    plsc.store_scatter(s_tbl, [loc_s], csort, mask=win)
    return 0
  lax.fori_loop(0, NVREG, scan_body, 0, unroll=4)

  def ext_body(k, off):
    sv = s_tbl[pl.ds(k * LANES, LANES)]
    m = sv >= 0
    mi = m.astype(jnp.int32)
    cs = plsc.cumsum(mi)
    tgt = off + cs - mi
    pos = sv & POS_MASK
    node = (lax.shift_right_logical(sv, POS_BITS) << 5) | wid
    plsc.store_scatter(l_v, [tgt], pos, mask=m)
    plsc.store_scatter(t_v, [lax.shift_right_logical(tgt, 7), tgt & 127],
                       node, mask=m)
    return off + jnp.max(cs)
  n = lax.fori_loop(0, NLOCV, ext_body, jnp.int32(0), unroll=4)

  @pl.when(n > 0)
  def _():
    l0 = jnp.full((LANES,), l_v[pl.ds(0, LANES)][0], jnp.int32)
    t0 = jnp.full((LANES,), t_v[0, pl.ds(0, LANES)][0], jnp.int32)
    def pad_body(p, _):
      at = n + p * LANES + ii
      plsc.store_scatter(l_v, [at], l0)
      plsc.store_scatter(t_v, [lax.shift_right_logical(at, 7), at & 127], t0)
      return 0
    lax.fori_loop(0, GROUP // LANES, pad_body, 0, unroll=8)

  nv[...] = n + jnp.zeros((LANES,), jnp.int32)
  pltpu.sync_copy(nv, nn_hbm.at[pl.ds(wid * CHUNK, LANES)])
  pltpu.sync_copy(l_v, lv_hbm.at[pl.ds(wid * CAP, CAP)])
  pltpu.sync_copy(t_v, tv_hbm.at[pl.ds(wid * NCH_MAX, NCH_MAX)])

  for d in descs:
    d.wait()
  pltpu.sync_copy(rows, m1g_hbm.at[pl.ds(wid * B_PER_W, B_PER_W)])


def _k1(idx, tbl):
  return pl.kernel(
      _k1_body,
      out_type=[
          jax.ShapeDtypeStruct((BATCH, PADW), jnp.float32),
          jax.ShapeDtypeStruct((NW * CAP,), jnp.int32),
          jax.ShapeDtypeStruct((NW * NCH_MAX, CHUNK), jnp.int32),
          jax.ShapeDtypeStruct((NW * CHUNK,), jnp.int32),
      ],
      mesh=_mesh(),
      compiler_params=_SC_PARAMS,
      scratch_types=[
          pltpu.VMEM((B_PER_W,), jnp.int32),
          pltpu.VMEM((B_PER_W, PADW), jnp.float32),
          pltpu.VMEM((BATCH,), jnp.int32),
          pltpu.VMEM((LOCAL_PAD,), jnp.int32),
          pltpu.VMEM((CAP,), jnp.int32),
          pltpu.VMEM((NCH_MAX, CHUNK), jnp.int32),
          pltpu.VMEM((LANES,), jnp.int32),
          pltpu.SemaphoreType.DMA,
      ],
  )(idx, tbl)


BB = 2048


def _k2_body(x_ref, dt_ref, m1_ref, wtune_ref, btune_ref, scale_ref,
             wdt_ref, bdt_ref, alog_ref, wb1_ref, bb1_ref, out_ref):
  x = x_ref[...]
  m1 = m1_ref[...][:, :HIDDEN]
  wt = wtune_ref[...][:, :INPUT_DIM]
  msg = lax.dot_general(x, wt, (((1,), (1,)), ((), ())),
                        preferred_element_type=jnp.float32) + btune_ref[...]
  upd = lax.dot_general(msg, wb1_ref[...], (((1,), (1,)), ((), ())),
                        preferred_element_type=jnp.float32) + bb1_ref[...]
  norm = jnp.sqrt(jnp.sum(m1 * m1, axis=-1, keepdims=True))
  rms = norm / (HIDDEN ** 0.5)
  h = scale_ref[...] * (m1 / (rms + 1e-8))
  z = jnp.sum(h * wdt_ref[...], axis=-1, keepdims=True) + bdt_ref[0, 0]
  dt = jax.nn.softplus(z) * dt_ref[...]
  decay = jnp.exp(-jnp.exp(alog_ref[...]) * dt)
  out_ref[:, :HIDDEN] = decay * m1 + dt * upd
  out_ref[:, HIDDEN:] = jnp.zeros((BB, PADW - HIDDEN), jnp.float32)


def _k2(x, delta_t, m1g, W_tune, b_tune, scale, W_dt, b_dt, A_log_1,
        W_B1, b_B1):
  grid = (BATCH // BB,)
  bs_b = lambda shape: pl.BlockSpec(shape, lambda i: (i, 0))
  bs_w = lambda shape: pl.BlockSpec(shape, lambda i: (0, 0))
  return pl.pallas_call(
      _k2_body,
      grid=grid,
      in_specs=[
          bs_b((BB, INPUT_DIM)),
          bs_b((BB, 1)),
          bs_b((BB, PADW)),
          bs_w((HIDDEN, INPUT_DIM + 2)),
          bs_w((1, HIDDEN)),
          bs_w((1, HIDDEN)),
          bs_w((1, HIDDEN)),
          bs_w((1, 1)),
          bs_w((1, HIDDEN)),
          bs_w((HIDDEN, HIDDEN)),
          bs_w((1, HIDDEN)),
      ],
      out_specs=bs_b((BB, PADW)),
      out_shape=jax.ShapeDtypeStruct((BATCH, PADW), jnp.float32),
  )(x, delta_t.reshape(BATCH, 1), m1g, W_tune, b_tune.reshape(1, HIDDEN),
    scale.reshape(1, HIDDEN), W_dt, b_dt.reshape(1, 1),
    A_log_1.reshape(1, HIDDEN), W_B1, b_B1.reshape(1, HIDDEN))


def _k3_body(tbl_hbm, newm1_hbm, lv_hbm, tv_hbm, nn_hbm, out_hbm,
             l_v, t_v, nv, b0, b1, b2, b3, g0, g1, g2, g3, ssem):
  del tbl_hbm
  wid = lax.axis_index("s") * 2 + lax.axis_index("c")

  pltpu.sync_copy(lv_hbm.at[pl.ds(wid * CAP, CAP)], l_v)
  pltpu.sync_copy(tv_hbm.at[pl.ds(wid * NCH_MAX, NCH_MAX)], t_v)
  pltpu.sync_copy(nn_hbm.at[pl.ds(wid * CHUNK, LANES)], nv)
  n = nv[pl.ds(0, LANES)][0]

  bufs = (b0, b1, b2, b3)
  gsems = (g0, g1, g2, g3)
  nrounds = (n + GROUP - 1) // GROUP

  def round_body(q, _):
    gds = []
    for b in range(NBUF):
      c = q * NBUF + b
      gds.append(pltpu.async_copy(
          newm1_hbm.at[l_v.at[pl.ds(c * CHUNK, CHUNK)]], bufs[b], gsems[b]))
    sds = []
    for b in range(NBUF):
      gds[b].wait()
      sds.append(pltpu.async_copy(bufs[b], out_hbm.at[t_v.at[q * NBUF + b]],
                                  ssem))
    for b in range(NBUF):
      sds[b].wait()
    return 0
  lax.fori_loop(0, nrounds, round_body, 0)


def _k3(tbl, new_m1, lv, tv, nn):
  return _mpmd._mpmd_map(
      [(_mesh(), _k3_body)],
      jax.ShapeDtypeStruct((NUM_NODES, PADW), jnp.float32),
      input_output_aliases={0: 0},
      compiler_params=_SC_PARAMS,
      scratch_types=[
          pltpu.VMEM((CAP,), jnp.int32),
          pltpu.VMEM((NCH_MAX, CHUNK), jnp.int32),
          pltpu.VMEM((LANES,), jnp.int32),
          pltpu.VMEM((CHUNK, PADW), jnp.float32),
          pltpu.VMEM((CHUNK, PADW), jnp.float32),
          pltpu.VMEM((CHUNK, PADW), jnp.float32),
          pltpu.VMEM((CHUNK, PADW), jnp.float32),
          pltpu.SemaphoreType.DMA,
          pltpu.SemaphoreType.DMA,
          pltpu.SemaphoreType.DMA,
          pltpu.SemaphoreType.DMA,
          pltpu.SemaphoreType.DMA,
      ],
  )(tbl, new_m1, lv, tv, nn)


def kernel(x, delta_t, idx, m1_vec, W_tune, b_tune, scale, W_dt, b_dt,
           A_log_1, W_B1, b_B1):
  idx = idx.astype(jnp.int32)
  padded = jnp.pad(m1_vec, ((0, 0), (0, PADW - HIDDEN)))
  m1g, lv, tv, nn = _k1(idx, padded)
  new_m1 = _k2(x, delta_t, m1g, W_tune, b_tune, scale, W_dt, b_dt,
               A_log_1, W_B1, b_B1)
  out_p = _k3(padded, new_m1, lv, tv, nn)
  return out_p[:, :HIDDEN]


# speedup vs baseline: 1.1062x; 1.1062x over previous
import jax
import jax.numpy as jnp
from jax import lax
from jax.experimental import pallas as pl
from jax.experimental.pallas import tpu as pltpu
from jax.experimental.pallas import tpu_sc as plsc
from jax._src.pallas import mpmd as _mpmd

NUM_NODES = 100000
HIDDEN = 64
INPUT_DIM = 128
BATCH = 16384
PADW = 128

NW = 32
LANES = 16
B_PER_W = BATCH // NW
NVREG = BATCH // LANES
LOCAL_PAD = 3136
NLOCV = LOCAL_PAD // LANES
POS_BITS = 14
POS_MASK = (1 << POS_BITS) - 1
SENTINEL = 2**31 - 1
CHUNK = 128
NBUF = 4
GROUP = NBUF * CHUNK
NCH_MAX = 32
CAP = NCH_MAX * CHUNK


def _shift_up(v):
  ii = lax.iota(jnp.int32, LANES)
  ind = jnp.minimum(ii + 1, LANES - 1)
  return lax.gather(
      v, ind[:, None],
      dimension_numbers=lax.GatherDimensionNumbers(
          offset_dims=(), collapsed_slice_dims=(0,), start_index_map=(0,)),
      slice_sizes=(1,),
      mode=lax.GatherScatterMode.PROMISE_IN_BOUNDS)


_SC_PARAMS = pltpu.CompilerParams(needs_layout_passes=False,
                                  use_tc_tiling_on_sc=True)


def _mesh():
  return plsc.VectorSubcoreMesh(core_axis_name="c", subcore_axis_name="s",
                                num_cores=2, num_subcores=16)


def _k1_body(idx_hbm, tbl_hbm, m1g_hbm, lv_hbm, tv_hbm, nn_hbm,
             idxv, rows, idx_all, s_tbl, l_v, t_v, nv, gsem):
  wid = lax.axis_index("s") * 2 + lax.axis_index("c")
  ii = lax.iota(jnp.int32, LANES)

  pltpu.sync_copy(idx_hbm.at[pl.ds(wid * B_PER_W, B_PER_W)], idxv)
  descs = []
  for k in range(B_PER_W // CHUNK):
    descs.append(pltpu.async_copy(
        tbl_hbm.at[idxv.at[pl.ds(k * CHUNK, CHUNK)]],
        rows.at[pl.ds(k * CHUNK, CHUNK)], gsem))

  pltpu.sync_copy(idx_hbm, idx_all)

  neg1 = jnp.full((LANES,), -1, jnp.int32)
  def init_body(k, _):
    s_tbl[pl.ds(k * LANES, LANES)] = neg1
    return 0
  lax.fori_loop(0, NLOCV, init_body, 0, unroll=8)

  def scan_body(j, _):
    v = idx_all[pl.ds(j * LANES, LANES)]
    mask = (v & (NW - 1)) == wid
    local = lax.shift_right_logical(v, 5)
    pos = j * LANES + ii
    comb = jnp.where(mask, (local << POS_BITS) | pos, SENTINEL)
    csort, _ = plsc.sort_key_val(comb, comb)
    nxt = _shift_up(csort)
    loc_s = lax.shift_right_logical(csort, POS_BITS)
    nxt_s = lax.shift_right_logical(nxt, POS_BITS)
    win = ((loc_s != nxt_s) | (ii == LANES - 1)) & (csort != SENTINEL)
    plsc.store_scatter(s_tbl, [loc_s], csort, mask=win)
    return 0
  lax.fori_loop(0, NVREG, scan_body, 0, unroll=4)

  def ext_body(k, off):
    sv = s_tbl[pl.ds(k * LANES, LANES)]
    m = sv >= 0
    mi = m.astype(jnp.int32)
    cs = plsc.cumsum(mi)
    tgt = off + cs - mi
    pos = sv & POS_MASK
    node = (lax.shift_right_logical(sv, POS_BITS) << 5) | wid
    plsc.store_scatter(l_v, [tgt], pos, mask=m)
    plsc.store_scatter(t_v, [lax.shift_right_logical(tgt, 7), tgt & 127],
                       node, mask=m)
    return off + jnp.max(cs)
  n = lax.fori_loop(0, NLOCV, ext_body, jnp.int32(0), unroll=4)

  @pl.when(n > 0)
  def _():
    l0 = jnp.full((LANES,), l_v[pl.ds(0, LANES)][0], jnp.int32)
    t0 = jnp.full((LANES,), t_v[0, pl.ds(0, LANES)][0], jnp.int32)
    def pad_body(p, _):
      at = n + p * LANES + ii
      plsc.store_scatter(l_v, [at], l0)
      plsc.store_scatter(t_v, [lax.shift_right_logical(at, 7), at & 127], t0)
      return 0
    lax.fori_loop(0, GROUP // LANES, pad_body, 0, unroll=8)

  nv[...] = n + jnp.zeros((LANES,), jnp.int32)
  pltpu.sync_copy(nv, nn_hbm.at[pl.ds(wid * CHUNK, LANES)])
  pltpu.sync_copy(l_v, lv_hbm.at[pl.ds(wid * CAP, CAP)])
  pltpu.sync_copy(t_v, tv_hbm.at[pl.ds(wid * NCH_MAX, NCH_MAX)])

  for d in descs:
    d.wait()
  pltpu.sync_copy(rows, m1g_hbm.at[pl.ds(wid * B_PER_W, B_PER_W)])


def _k1(idx, tbl):
  return pl.kernel(
      _k1_body,
      out_type=[
          jax.ShapeDtypeStruct((BATCH, PADW), jnp.float32),
          jax.ShapeDtypeStruct((NW * CAP,), jnp.int32),
          jax.ShapeDtypeStruct((NW * NCH_MAX, CHUNK), jnp.int32),
          jax.ShapeDtypeStruct((NW * CHUNK,), jnp.int32),
      ],
      mesh=_mesh(),
      compiler_params=_SC_PARAMS,
      scratch_types=[
          pltpu.VMEM((B_PER_W,), jnp.int32),
          pltpu.VMEM((B_PER_W, PADW), jnp.float32),
          pltpu.VMEM((BATCH,), jnp.int32),
          pltpu.VMEM((LOCAL_PAD,), jnp.int32),
          pltpu.VMEM((CAP,), jnp.int32),
          pltpu.VMEM((NCH_MAX, CHUNK), jnp.int32),
          pltpu.VMEM((LANES,), jnp.int32),
          pltpu.SemaphoreType.DMA,
      ],
  )(idx, tbl)


TB = 5120


def _kpad_body(mt_ref, out_ref):
  out_ref[:, :HIDDEN] = mt_ref[...].T
  out_ref[:, HIDDEN:] = jnp.zeros((TB, PADW - HIDDEN), jnp.float32)


def _kpad(m1_vec):
  return pl.pallas_call(
      _kpad_body,
      grid=(pl.cdiv(NUM_NODES, TB),),
      in_specs=[pl.BlockSpec((HIDDEN, TB), lambda i: (0, i))],
      out_specs=pl.BlockSpec((TB, PADW), lambda i: (i, 0)),
      out_shape=jax.ShapeDtypeStruct((NUM_NODES, PADW), jnp.float32),
  )(m1_vec.T)


def _kout_body(in_ref, out_ref):
  out_ref[...] = in_ref[...][:, :HIDDEN].T


def _kout(out_p):
  return pl.pallas_call(
      _kout_body,
      grid=(pl.cdiv(NUM_NODES, TB),),
      in_specs=[pl.BlockSpec((TB, PADW), lambda i: (i, 0))],
      out_specs=pl.BlockSpec((HIDDEN, TB), lambda i: (0, i)),
      out_shape=jax.ShapeDtypeStruct((HIDDEN, NUM_NODES), jnp.float32),
  )(out_p)


BB = 2048


def _k2_body(x_ref, dt_ref, m1_ref, wtune_ref, btune_ref, scale_ref,
             wdt_ref, bdt_ref, alog_ref, wb1_ref, bb1_ref, out_ref):
  x = x_ref[...]
  m1 = m1_ref[...][:, :HIDDEN]
  wt = wtune_ref[...][:, :INPUT_DIM]
  msg = lax.dot_general(x, wt, (((1,), (1,)), ((), ())),
                        preferred_element_type=jnp.float32) + btune_ref[...]
  upd = lax.dot_general(msg, wb1_ref[...], (((1,), (1,)), ((), ())),
                        preferred_element_type=jnp.float32) + bb1_ref[...]
  norm = jnp.sqrt(jnp.sum(m1 * m1, axis=-1, keepdims=True))
  rms = norm / (HIDDEN ** 0.5)
  h = scale_ref[...] * (m1 / (rms + 1e-8))
  z = jnp.sum(h * wdt_ref[...], axis=-1, keepdims=True) + bdt_ref[0, 0]
  dt = jax.nn.softplus(z) * dt_ref[...]
  decay = jnp.exp(-jnp.exp(alog_ref[...]) * dt)
  out_ref[:, :HIDDEN] = decay * m1 + dt * upd
  out_ref[:, HIDDEN:] = jnp.zeros((BB, PADW - HIDDEN), jnp.float32)


def _k2(x, delta_t, m1g, W_tune, b_tune, scale, W_dt, b_dt, A_log_1,
        W_B1, b_B1):
  grid = (BATCH // BB,)
  bs_b = lambda shape: pl.BlockSpec(shape, lambda i: (i, 0))
  bs_w = lambda shape: pl.BlockSpec(shape, lambda i: (0, 0))
  return pl.pallas_call(
      _k2_body,
      grid=grid,
      in_specs=[
          bs_b((BB, INPUT_DIM)),
          bs_b((BB, 1)),
          bs_b((BB, PADW)),
          bs_w((HIDDEN, INPUT_DIM + 2)),
          bs_w((1, HIDDEN)),
          bs_w((1, HIDDEN)),
          bs_w((1, HIDDEN)),
          bs_w((1, 1)),
          bs_w((1, HIDDEN)),
          bs_w((HIDDEN, HIDDEN)),
          bs_w((1, HIDDEN)),
      ],
      out_specs=bs_b((BB, PADW)),
      out_shape=jax.ShapeDtypeStruct((BATCH, PADW), jnp.float32),
  )(x, delta_t.reshape(BATCH, 1), m1g, W_tune, b_tune.reshape(1, HIDDEN),
    scale.reshape(1, HIDDEN), W_dt, b_dt.reshape(1, 1),
    A_log_1.reshape(1, HIDDEN), W_B1, b_B1.reshape(1, HIDDEN))


def _k3_body(tbl_hbm, newm1_hbm, lv_hbm, tv_hbm, nn_hbm, out_hbm,
             l_v, t_v, nv, b0, b1, b2, b3, g0, g1, g2, g3, ssem):
  del tbl_hbm
  wid = lax.axis_index("s") * 2 + lax.axis_index("c")

  pltpu.sync_copy(lv_hbm.at[pl.ds(wid * CAP, CAP)], l_v)
  pltpu.sync_copy(tv_hbm.at[pl.ds(wid * NCH_MAX, NCH_MAX)], t_v)
  pltpu.sync_copy(nn_hbm.at[pl.ds(wid * CHUNK, LANES)], nv)
  n = nv[pl.ds(0, LANES)][0]

  bufs = (b0, b1, b2, b3)
  gsems = (g0, g1, g2, g3)
  nrounds = (n + GROUP - 1) // GROUP

  def round_body(q, _):
    gds = []
    for b in range(NBUF):
      c = q * NBUF + b
      gds.append(pltpu.async_copy(
          newm1_hbm.at[l_v.at[pl.ds(c * CHUNK, CHUNK)]], bufs[b], gsems[b]))
    sds = []
    for b in range(NBUF):
      gds[b].wait()
      sds.append(pltpu.async_copy(bufs[b], out_hbm.at[t_v.at[q * NBUF + b]],
                                  ssem))
    for b in range(NBUF):
      sds[b].wait()
    return 0
  lax.fori_loop(0, nrounds, round_body, 0)


def _k3(tbl, new_m1, lv, tv, nn):
  return _mpmd._mpmd_map(
      [(_mesh(), _k3_body)],
      jax.ShapeDtypeStruct((NUM_NODES, PADW), jnp.float32),
      input_output_aliases={0: 0},
      compiler_params=_SC_PARAMS,
      scratch_types=[
          pltpu.VMEM((CAP,), jnp.int32),
          pltpu.VMEM((NCH_MAX, CHUNK), jnp.int32),
          pltpu.VMEM((LANES,), jnp.int32),
          pltpu.VMEM((CHUNK, PADW), jnp.float32),
          pltpu.VMEM((CHUNK, PADW), jnp.float32),
          pltpu.VMEM((CHUNK, PADW), jnp.float32),
          pltpu.VMEM((CHUNK, PADW), jnp.float32),
          pltpu.SemaphoreType.DMA,
          pltpu.SemaphoreType.DMA,
          pltpu.SemaphoreType.DMA,
          pltpu.SemaphoreType.DMA,
          pltpu.SemaphoreType.DMA,
      ],
  )(tbl, new_m1, lv, tv, nn)


def kernel(x, delta_t, idx, m1_vec, W_tune, b_tune, scale, W_dt, b_dt,
           A_log_1, W_B1, b_B1):
  idx = idx.astype(jnp.int32)
  padded = _kpad(m1_vec)
  m1g, lv, tv, nn = _k1(idx, padded)
  new_m1 = _k2(x, delta_t, m1g, W_tune, b_tune, scale, W_dt, b_dt,
               A_log_1, W_B1, b_B1)
  out_p = _k3(padded, new_m1, lv, tv, nn)
  return _kout(out_p).T


# speedup vs baseline: 1.1712x; 1.0588x over previous
import jax
import jax.numpy as jnp
from jax import lax
from jax.experimental import pallas as pl
from jax.experimental.pallas import tpu as pltpu
from jax.experimental.pallas import tpu_sc as plsc
from jax._src.pallas import mpmd as _mpmd

NUM_NODES = 100000
HIDDEN = 64
INPUT_DIM = 128
BATCH = 16384
PADW = 128

NW = 32
LANES = 16
B_PER_W = BATCH // NW
NVREG = BATCH // LANES
LOCAL_PAD = 3136
NLOCV = LOCAL_PAD // LANES
POS_BITS = 14
POS_MASK = (1 << POS_BITS) - 1
SENTINEL = 2**31 - 1
CHUNK = 128
NBUF = 4
GROUP = NBUF * CHUNK
NCH_MAX = 32
CAP = NCH_MAX * CHUNK


def _shift_up(v):
  ii = lax.iota(jnp.int32, LANES)
  ind = jnp.minimum(ii + 1, LANES - 1)
  return lax.gather(
      v, ind[:, None],
      dimension_numbers=lax.GatherDimensionNumbers(
          offset_dims=(), collapsed_slice_dims=(0,), start_index_map=(0,)),
      slice_sizes=(1,),
      mode=lax.GatherScatterMode.PROMISE_IN_BOUNDS)


_SC_PARAMS = pltpu.CompilerParams(needs_layout_passes=False,
                                  use_tc_tiling_on_sc=True)


def _mesh():
  return plsc.VectorSubcoreMesh(core_axis_name="c", subcore_axis_name="s",
                                num_cores=2, num_subcores=16)


def _k0_body(idx_hbm, lv_hbm, tv_hbm, nn_hbm,
             idx_all, s_tbl, l_v, t_v, nv):
  wid = lax.axis_index("s") * 2 + lax.axis_index("c")
  ii = lax.iota(jnp.int32, LANES)

  pltpu.sync_copy(idx_hbm, idx_all)

  neg1 = jnp.full((LANES,), -1, jnp.int32)
  def init_body(k, _):
    s_tbl[pl.ds(k * LANES, LANES)] = neg1
    return 0
  lax.fori_loop(0, NLOCV, init_body, 0, unroll=8)

  def scan_body(j, _):
    v = idx_all[pl.ds(j * LANES, LANES)]
    mask = (v & (NW - 1)) == wid
    local = lax.shift_right_logical(v, 5)
    pos = j * LANES + ii
    comb = jnp.where(mask, (local << POS_BITS) | pos, SENTINEL)
    csort, _ = plsc.sort_key_val(comb, comb)
    nxt = _shift_up(csort)
    loc_s = lax.shift_right_logical(csort, POS_BITS)
    nxt_s = lax.shift_right_logical(nxt, POS_BITS)
    win = ((loc_s != nxt_s) | (ii == LANES - 1)) & (csort != SENTINEL)
    plsc.store_scatter(s_tbl, [loc_s], csort, mask=win)
    return 0
  lax.fori_loop(0, NVREG, scan_body, 0, unroll=4)

  def ext_body(k, off):
    sv = s_tbl[pl.ds(k * LANES, LANES)]
    m = sv >= 0
    mi = m.astype(jnp.int32)
    cs = plsc.cumsum(mi)
    tgt = off + cs - mi
    pos = sv & POS_MASK
    node = (lax.shift_right_logical(sv, POS_BITS) << 5) | wid
    plsc.store_scatter(l_v, [tgt], pos, mask=m)
    plsc.store_scatter(t_v, [lax.shift_right_logical(tgt, 7), tgt & 127],
                       node, mask=m)
    return off + jnp.max(cs)
  n = lax.fori_loop(0, NLOCV, ext_body, jnp.int32(0), unroll=4)

  @pl.when(n > 0)
  def _():
    l0 = jnp.full((LANES,), l_v[pl.ds(0, LANES)][0], jnp.int32)
    t0 = jnp.full((LANES,), t_v[0, pl.ds(0, LANES)][0], jnp.int32)
    def pad_body(p, _):
      at = n + p * LANES + ii
      plsc.store_scatter(l_v, [at], l0)
      plsc.store_scatter(t_v, [lax.shift_right_logical(at, 7), at & 127], t0)
      return 0
    lax.fori_loop(0, GROUP // LANES, pad_body, 0, unroll=8)

  nv[...] = n + jnp.zeros((LANES,), jnp.int32)
  pltpu.sync_copy(nv, nn_hbm.at[pl.ds(wid * CHUNK, LANES)])
  pltpu.sync_copy(l_v, lv_hbm.at[pl.ds(wid * CAP, CAP)])
  pltpu.sync_copy(t_v, tv_hbm.at[pl.ds(wid * NCH_MAX, NCH_MAX)])


def _k0(idx):
  return pl.kernel(
      _k0_body,
      out_type=[
          jax.ShapeDtypeStruct((NW * CAP,), jnp.int32),
          jax.ShapeDtypeStruct((NW * NCH_MAX, CHUNK), jnp.int32),
          jax.ShapeDtypeStruct((NW * CHUNK,), jnp.int32),
      ],
      mesh=_mesh(),
      compiler_params=_SC_PARAMS,
      scratch_types=[
          pltpu.VMEM((BATCH,), jnp.int32),
          pltpu.VMEM((LOCAL_PAD,), jnp.int32),
          pltpu.VMEM((CAP,), jnp.int32),
          pltpu.VMEM((NCH_MAX, CHUNK), jnp.int32),
          pltpu.VMEM((LANES,), jnp.int32),
      ],
  )(idx)


def _k1_body(idx_hbm, tbl_hbm, m1g_hbm, idxv, rows, gsem):
  wid = lax.axis_index("s") * 2 + lax.axis_index("c")
  pltpu.sync_copy(idx_hbm.at[pl.ds(wid * B_PER_W, B_PER_W)], idxv)
  descs = []
  for k in range(B_PER_W // CHUNK):
    descs.append(pltpu.async_copy(
        tbl_hbm.at[idxv.at[pl.ds(k * CHUNK, CHUNK)]],
        rows.at[pl.ds(k * CHUNK, CHUNK)], gsem))
  for d in descs:
    d.wait()
  pltpu.sync_copy(rows, m1g_hbm.at[pl.ds(wid * B_PER_W, B_PER_W)])


def _k1(idx, tbl):
  return pl.kernel(
      _k1_body,
      out_type=jax.ShapeDtypeStruct((BATCH, PADW), jnp.float32),
      mesh=_mesh(),
      compiler_params=_SC_PARAMS,
      scratch_types=[
          pltpu.VMEM((B_PER_W,), jnp.int32),
          pltpu.VMEM((B_PER_W, PADW), jnp.float32),
          pltpu.SemaphoreType.DMA,
      ],
  )(idx, tbl)


TB = 5120


def _kpad_body(mt_ref, out_ref):
  out_ref[:, :HIDDEN] = mt_ref[...].T
  out_ref[:, HIDDEN:] = jnp.zeros((TB, PADW - HIDDEN), jnp.float32)


def _kpad(m1_vec):
  return pl.pallas_call(
      _kpad_body,
      grid=(pl.cdiv(NUM_NODES, TB),),
      in_specs=[pl.BlockSpec((HIDDEN, TB), lambda i: (0, i))],
      out_specs=pl.BlockSpec((TB, PADW), lambda i: (i, 0)),
      out_shape=jax.ShapeDtypeStruct((NUM_NODES, PADW), jnp.float32),
  )(m1_vec.T)


def _kout_body(in_ref, out_ref):
  out_ref[...] = in_ref[...][:, :HIDDEN].T


def _kout(out_p):
  return pl.pallas_call(
      _kout_body,
      grid=(pl.cdiv(NUM_NODES, TB),),
      in_specs=[pl.BlockSpec((TB, PADW), lambda i: (i, 0))],
      out_specs=pl.BlockSpec((HIDDEN, TB), lambda i: (0, i)),
      out_shape=jax.ShapeDtypeStruct((HIDDEN, NUM_NODES), jnp.float32),
  )(out_p)


BB = 2048


def _k2_body(x_ref, dt_ref, m1_ref, wtune_ref, btune_ref, scale_ref,
             wdt_ref, bdt_ref, alog_ref, wb1_ref, bb1_ref, out_ref):
  x = x_ref[...]
  m1 = m1_ref[...][:, :HIDDEN]
  wt = wtune_ref[...][:, :INPUT_DIM]
  msg = lax.dot_general(x, wt, (((1,), (1,)), ((), ())),
                        preferred_element_type=jnp.float32) + btune_ref[...]
  upd = lax.dot_general(msg, wb1_ref[...], (((1,), (1,)), ((), ())),
                        preferred_element_type=jnp.float32) + bb1_ref[...]
  norm = jnp.sqrt(jnp.sum(m1 * m1, axis=-1, keepdims=True))
  rms = norm / (HIDDEN ** 0.5)
  h = scale_ref[...] * (m1 / (rms + 1e-8))
  z = jnp.sum(h * wdt_ref[...], axis=-1, keepdims=True) + bdt_ref[0, 0]
  dt = jax.nn.softplus(z) * dt_ref[...]
  decay = jnp.exp(-jnp.exp(alog_ref[...]) * dt)
  out_ref[:, :HIDDEN] = decay * m1 + dt * upd
  out_ref[:, HIDDEN:] = jnp.zeros((BB, PADW - HIDDEN), jnp.float32)


def _k2(x, delta_t, m1g, W_tune, b_tune, scale, W_dt, b_dt, A_log_1,
        W_B1, b_B1):
  grid = (BATCH // BB,)
  bs_b = lambda shape: pl.BlockSpec(shape, lambda i: (i, 0))
  bs_w = lambda shape: pl.BlockSpec(shape, lambda i: (0, 0))
  return pl.pallas_call(
      _k2_body,
      grid=grid,
      in_specs=[
          bs_b((BB, INPUT_DIM)),
          bs_b((BB, 1)),
          bs_b((BB, PADW)),
          bs_w((HIDDEN, INPUT_DIM + 2)),
          bs_w((1, HIDDEN)),
          bs_w((1, HIDDEN)),
          bs_w((1, HIDDEN)),
          bs_w((1, 1)),
          bs_w((1, HIDDEN)),
          bs_w((HIDDEN, HIDDEN)),
          bs_w((1, HIDDEN)),
      ],
      out_specs=bs_b((BB, PADW)),
      out_shape=jax.ShapeDtypeStruct((BATCH, PADW), jnp.float32),
  )(x, delta_t.reshape(BATCH, 1), m1g, W_tune, b_tune.reshape(1, HIDDEN),
    scale.reshape(1, HIDDEN), W_dt, b_dt.reshape(1, 1),
    A_log_1.reshape(1, HIDDEN), W_B1, b_B1.reshape(1, HIDDEN))


def _k3_body(tbl_hbm, newm1_hbm, lv_hbm, tv_hbm, nn_hbm, out_hbm,
             l_v, t_v, nv, b0, b1, b2, b3, g0, g1, g2, g3, ssem):
  del tbl_hbm
  wid = lax.axis_index("s") * 2 + lax.axis_index("c")

  pltpu.sync_copy(lv_hbm.at[pl.ds(wid * CAP, CAP)], l_v)
  pltpu.sync_copy(tv_hbm.at[pl.ds(wid * NCH_MAX, NCH_MAX)], t_v)
  pltpu.sync_copy(nn_hbm.at[pl.ds(wid * CHUNK, LANES)], nv)
  n = nv[pl.ds(0, LANES)][0]

  bufs = (b0, b1, b2, b3)
  gsems = (g0, g1, g2, g3)
  nrounds = (n + GROUP - 1) // GROUP

  def round_body(q, _):
    gds = []
    for b in range(NBUF):
      c = q * NBUF + b
      gds.append(pltpu.async_copy(
          newm1_hbm.at[l_v.at[pl.ds(c * CHUNK, CHUNK)]], bufs[b], gsems[b]))
    sds = []
    for b in range(NBUF):
      gds[b].wait()
      sds.append(pltpu.async_copy(bufs[b], out_hbm.at[t_v.at[q * NBUF + b]],
                                  ssem))
    for b in range(NBUF):
      sds[b].wait()
    return 0
  lax.fori_loop(0, nrounds, round_body, 0)


def _k3(tbl, new_m1, lv, tv, nn):
  return _mpmd._mpmd_map(
      [(_mesh(), _k3_body)],
      jax.ShapeDtypeStruct((NUM_NODES, PADW), jnp.float32),
      input_output_aliases={0: 0},
      compiler_params=_SC_PARAMS,
      scratch_types=[
          pltpu.VMEM((CAP,), jnp.int32),
          pltpu.VMEM((NCH_MAX, CHUNK), jnp.int32),
          pltpu.VMEM((LANES,), jnp.int32),
          pltpu.VMEM((CHUNK, PADW), jnp.float32),
          pltpu.VMEM((CHUNK, PADW), jnp.float32),
          pltpu.VMEM((CHUNK, PADW), jnp.float32),
          pltpu.VMEM((CHUNK, PADW), jnp.float32),
          pltpu.SemaphoreType.DMA,
          pltpu.SemaphoreType.DMA,
          pltpu.SemaphoreType.DMA,
          pltpu.SemaphoreType.DMA,
          pltpu.SemaphoreType.DMA,
      ],
  )(tbl, new_m1, lv, tv, nn)


def kernel(x, delta_t, idx, m1_vec, W_tune, b_tune, scale, W_dt, b_dt,
           A_log_1, W_B1, b_B1):
  idx = idx.astype(jnp.int32)
  lv, tv, nn = _k0(idx)
  padded = _kpad(m1_vec)
  m1g = _k1(idx, padded)
  new_m1 = _k2(x, delta_t, m1g, W_tune, b_tune, scale, W_dt, b_dt,
               A_log_1, W_B1, b_B1)
  out_p = _k3(padded, new_m1, lv, tv, nn)
  return _kout(out_p).T


# speedup vs baseline: 1.2743x; 1.0880x over previous
import jax
import jax.numpy as jnp
from jax import lax
from jax.experimental import pallas as pl
from jax.experimental.pallas import tpu as pltpu
from jax.experimental.pallas import tpu_sc as plsc
from jax._src.pallas import mpmd as _mpmd

NUM_NODES = 100000
HIDDEN = 64
INPUT_DIM = 128
BATCH = 16384
PADW = 128

NW = 32
LANES = 16
B_PER_W = BATCH // NW
NVREG = BATCH // LANES
LOCAL_PAD = 3136
NLOCV = LOCAL_PAD // LANES
POS_BITS = 14
POS_MASK = (1 << POS_BITS) - 1
SENTINEL = 2**31 - 1
CHUNK = 128
NBUF = 4
GROUP = NBUF * CHUNK
NCH_MAX = 32
CAP = NCH_MAX * CHUNK


def _shift_up(v):
  ii = lax.iota(jnp.int32, LANES)
  ind = jnp.minimum(ii + 1, LANES - 1)
  return lax.gather(
      v, ind[:, None],
      dimension_numbers=lax.GatherDimensionNumbers(
          offset_dims=(), collapsed_slice_dims=(0,), start_index_map=(0,)),
      slice_sizes=(1,),
      mode=lax.GatherScatterMode.PROMISE_IN_BOUNDS)


_SC_PARAMS = pltpu.CompilerParams(needs_layout_passes=False,
                                  use_tc_tiling_on_sc=True)


def _mesh():
  return plsc.VectorSubcoreMesh(core_axis_name="c", subcore_axis_name="s",
                                num_cores=2, num_subcores=16)


def _k0_body(idx_hbm, lv_hbm, tv_hbm, nn_hbm,
             idx_all, s_tbl, l_v, t_v, nv):
  wid = lax.axis_index("s") * 2 + lax.axis_index("c")
  ii = lax.iota(jnp.int32, LANES)

  pltpu.sync_copy(idx_hbm, idx_all)

  neg1 = jnp.full((LANES,), -1, jnp.int32)
  def init_body(k, _):
    s_tbl[pl.ds(k * LANES, LANES)] = neg1
    return 0
  lax.fori_loop(0, NLOCV, init_body, 0, unroll=8)

  def scan_body(j, _):
    v = idx_all[pl.ds(j * LANES, LANES)]
    mask = (v & (NW - 1)) == wid
    local = lax.shift_right_logical(v, 5)
    pos = j * LANES + ii
    comb = jnp.where(mask, (local << POS_BITS) | pos, SENTINEL)
    csort, _ = plsc.sort_key_val(comb, comb)
    nxt = _shift_up(csort)
    loc_s = lax.shift_right_logical(csort, POS_BITS)
    nxt_s = lax.shift_right_logical(nxt, POS_BITS)
    win = ((loc_s != nxt_s) | (ii == LANES - 1)) & (csort != SENTINEL)
    plsc.store_scatter(s_tbl, [loc_s], csort, mask=win)
    return 0
  lax.fori_loop(0, NVREG, scan_body, 0, unroll=4)

  def ext_body(k, off):
    sv = s_tbl[pl.ds(k * LANES, LANES)]
    m = sv >= 0
    mi = m.astype(jnp.int32)
    cs = plsc.cumsum(mi)
    tgt = off + cs - mi
    pos = sv & POS_MASK
    node = (lax.shift_right_logical(sv, POS_BITS) << 5) | wid
    plsc.store_scatter(l_v, [tgt], pos, mask=m)
    plsc.store_scatter(t_v, [lax.shift_right_logical(tgt, 7), tgt & 127],
                       node, mask=m)
    return off + jnp.max(cs)
  n = lax.fori_loop(0, NLOCV, ext_body, jnp.int32(0), unroll=4)

  @pl.when(n > 0)
  def _():
    l0 = jnp.full((LANES,), l_v[pl.ds(0, LANES)][0], jnp.int32)
    t0 = jnp.full((LANES,), t_v[0, pl.ds(0, LANES)][0], jnp.int32)
    def pad_body(p, _):
      at = n + p * LANES + ii
      plsc.store_scatter(l_v, [at], l0)
      plsc.store_scatter(t_v, [lax.shift_right_logical(at, 7), at & 127], t0)
      return 0
    lax.fori_loop(0, GROUP // LANES, pad_body, 0, unroll=8)

  nv[...] = n + jnp.zeros((LANES,), jnp.int32)
  pltpu.sync_copy(nv, nn_hbm.at[pl.ds(wid * CHUNK, LANES)])
  pltpu.sync_copy(l_v, lv_hbm.at[pl.ds(wid * CAP, CAP)])
  pltpu.sync_copy(t_v, tv_hbm.at[pl.ds(wid * NCH_MAX, NCH_MAX)])


def _k0(idx):
  return pl.kernel(
      _k0_body,
      out_type=[
          jax.ShapeDtypeStruct((NW * CAP,), jnp.int32),
          jax.ShapeDtypeStruct((NW * NCH_MAX, CHUNK), jnp.int32),
          jax.ShapeDtypeStruct((NW * CHUNK,), jnp.int32),
      ],
      mesh=_mesh(),
      compiler_params=_SC_PARAMS,
      scratch_types=[
          pltpu.VMEM((BATCH,), jnp.int32),
          pltpu.VMEM((LOCAL_PAD,), jnp.int32),
          pltpu.VMEM((CAP,), jnp.int32),
          pltpu.VMEM((NCH_MAX, CHUNK), jnp.int32),
          pltpu.VMEM((LANES,), jnp.int32),
      ],
  )(idx)


def _k1_body(idx_hbm, tbl_hbm, nn_hbm, m1g_hbm, idxv, rows, gsem):
  del nn_hbm
  wid = lax.axis_index("s") * 2 + lax.axis_index("c")
  pltpu.sync_copy(idx_hbm.at[pl.ds(wid * B_PER_W, B_PER_W)], idxv)
  descs = []
  for k in range(B_PER_W // CHUNK):
    descs.append(pltpu.async_copy(
        tbl_hbm.at[idxv.at[pl.ds(k * CHUNK, CHUNK)]],
        rows.at[pl.ds(k * CHUNK, CHUNK)], gsem))
  for d in descs:
    d.wait()
  pltpu.sync_copy(rows, m1g_hbm.at[pl.ds(wid * B_PER_W, B_PER_W)])


def _k1(idx, tbl, nn):
  return pl.kernel(
      _k1_body,
      out_type=jax.ShapeDtypeStruct((BATCH, PADW), jnp.float32),
      mesh=_mesh(),
      compiler_params=_SC_PARAMS,
      scratch_types=[
          pltpu.VMEM((B_PER_W,), jnp.int32),
          pltpu.VMEM((B_PER_W, PADW), jnp.float32),
          pltpu.SemaphoreType.DMA,
      ],
  )(idx, tbl, nn)


TB = 5120


def _kpad_body(mt_ref, out_ref):
  out_ref[:, :HIDDEN] = mt_ref[...].T
  out_ref[:, HIDDEN:] = jnp.zeros((TB, PADW - HIDDEN), jnp.float32)


def _kpad(m1_vec):
  return pl.pallas_call(
      _kpad_body,
      grid=(pl.cdiv(NUM_NODES, TB),),
      in_specs=[pl.BlockSpec((HIDDEN, TB), lambda i: (0, i))],
      out_specs=pl.BlockSpec((TB, PADW), lambda i: (i, 0)),
      out_shape=jax.ShapeDtypeStruct((NUM_NODES, PADW), jnp.float32),
  )(m1_vec.T)


def _kout_body(in_ref, out_ref):
  out_ref[...] = in_ref[...][:, :HIDDEN].T


def _kout(out_p):
  return pl.pallas_call(
      _kout_body,
      grid=(pl.cdiv(NUM_NODES, TB),),
      in_specs=[pl.BlockSpec((TB, PADW), lambda i: (i, 0))],
      out_specs=pl.BlockSpec((HIDDEN, TB), lambda i: (0, i)),
      out_shape=jax.ShapeDtypeStruct((HIDDEN, NUM_NODES), jnp.float32),
  )(out_p)


BB = 2048


def _k2_body(x_ref, dt_ref, m1_ref, wtune_ref, btune_ref, scale_ref,
             wdt_ref, bdt_ref, alog_ref, wb1_ref, bb1_ref, out_ref):
  x = x_ref[...]
  m1 = m1_ref[...][:, :HIDDEN]
  wt = wtune_ref[...][:, :INPUT_DIM]
  msg = lax.dot_general(x, wt, (((1,), (1,)), ((), ())),
                        preferred_element_type=jnp.float32) + btune_ref[...]
  upd = lax.dot_general(msg, wb1_ref[...], (((1,), (1,)), ((), ())),
                        preferred_element_type=jnp.float32) + bb1_ref[...]
  norm = jnp.sqrt(jnp.sum(m1 * m1, axis=-1, keepdims=True))
  rms = norm / (HIDDEN ** 0.5)
  h = scale_ref[...] * (m1 / (rms + 1e-8))
  z = jnp.sum(h * wdt_ref[...], axis=-1, keepdims=True) + bdt_ref[0, 0]
  dt = jax.nn.softplus(z) * dt_ref[...]
  decay = jnp.exp(-jnp.exp(alog_ref[...]) * dt)
  out_ref[:, :HIDDEN] = decay * m1 + dt * upd
  out_ref[:, HIDDEN:] = jnp.zeros((BB, PADW - HIDDEN), jnp.float32)


def _k2(x, delta_t, m1g, W_tune, b_tune, scale, W_dt, b_dt, A_log_1,
        W_B1, b_B1):
  grid = (BATCH // BB,)
  bs_b = lambda shape: pl.BlockSpec(shape, lambda i: (i, 0))
  bs_w = lambda shape: pl.BlockSpec(shape, lambda i: (0, 0))
  return pl.pallas_call(
      _k2_body,
      grid=grid,
      in_specs=[
          bs_b((BB, INPUT_DIM)),
          bs_b((BB, 1)),
          bs_b((BB, PADW)),
          bs_w((HIDDEN, INPUT_DIM + 2)),
          bs_w((1, HIDDEN)),
          bs_w((1, HIDDEN)),
          bs_w((1, HIDDEN)),
          bs_w((1, 1)),
          bs_w((1, HIDDEN)),
          bs_w((HIDDEN, HIDDEN)),
          bs_w((1, HIDDEN)),
      ],
      out_specs=bs_b((BB, PADW)),
      out_shape=jax.ShapeDtypeStruct((BATCH, PADW), jnp.float32),
  )(x, delta_t.reshape(BATCH, 1), m1g, W_tune, b_tune.reshape(1, HIDDEN),
    scale.reshape(1, HIDDEN), W_dt, b_dt.reshape(1, 1),
    A_log_1.reshape(1, HIDDEN), W_B1, b_B1.reshape(1, HIDDEN))


def _k3_body(tbl_hbm, newm1_hbm, lv_hbm, tv_hbm, nn_hbm, out_hbm,
             l_v, t_v, nv, b0, b1, b2, b3, g0, g1, g2, g3, ssem):
  del tbl_hbm
  wid = lax.axis_index("s") * 2 + lax.axis_index("c")

  pltpu.sync_copy(lv_hbm.at[pl.ds(wid * CAP, CAP)], l_v)
  pltpu.sync_copy(tv_hbm.at[pl.ds(wid * NCH_MAX, NCH_MAX)], t_v)
  pltpu.sync_copy(nn_hbm.at[pl.ds(wid * CHUNK, LANES)], nv)
  n = nv[pl.ds(0, LANES)][0]

  bufs = (b0, b1, b2, b3)
  gsems = (g0, g1, g2, g3)
  nrounds = (n + GROUP - 1) // GROUP

  def round_body(q, _):
    gds = []
    for b in range(NBUF):
      c = q * NBUF + b
      gds.append(pltpu.async_copy(
          newm1_hbm.at[l_v.at[pl.ds(c * CHUNK, CHUNK)]], bufs[b], gsems[b]))
    sds = []
    for b in range(NBUF):
      gds[b].wait()
      sds.append(pltpu.async_copy(bufs[b], out_hbm.at[t_v.at[q * NBUF + b]],
                                  ssem))
    for b in range(NBUF):
      sds[b].wait()
    return 0
  lax.fori_loop(0, nrounds, round_body, 0)


def _k3(tbl, new_m1, lv, tv, nn):
  return _mpmd._mpmd_map(
      [(_mesh(), _k3_body)],
      jax.ShapeDtypeStruct((NUM_NODES, PADW), jnp.float32),
      input_output_aliases={0: 0},
      compiler_params=_SC_PARAMS,
      scratch_types=[
          pltpu.VMEM((CAP,), jnp.int32),
          pltpu.VMEM((NCH_MAX, CHUNK), jnp.int32),
          pltpu.VMEM((LANES,), jnp.int32),
          pltpu.VMEM((CHUNK, PADW), jnp.float32),
          pltpu.VMEM((CHUNK, PADW), jnp.float32),
          pltpu.VMEM((CHUNK, PADW), jnp.float32),
          pltpu.VMEM((CHUNK, PADW), jnp.float32),
          pltpu.SemaphoreType.DMA,
          pltpu.SemaphoreType.DMA,
          pltpu.SemaphoreType.DMA,
          pltpu.SemaphoreType.DMA,
          pltpu.SemaphoreType.DMA,
      ],
  )(tbl, new_m1, lv, tv, nn)


def kernel(x, delta_t, idx, m1_vec, W_tune, b_tune, scale, W_dt, b_dt,
           A_log_1, W_B1, b_B1):
  idx = idx.astype(jnp.int32)
  lv, tv, nn = _k0(idx)
  padded = _kpad(m1_vec)
  m1g = _k1(idx, padded, nn)
  new_m1 = _k2(x, delta_t, m1g, W_tune, b_tune, scale, W_dt, b_dt,
               A_log_1, W_B1, b_B1)
  out_p = _k3(padded, new_m1, lv, tv, nn)
  return _kout(out_p).T
